# baseline (speedup 1.0000x reference)
import functools

import jax
import jax.numpy as jnp
from jax.experimental import pallas as pl
from jax.experimental.pallas import tpu as pltpu

_MAX_ANS_LEN = 30
_EPS = 1e-8
_NEG = -10000.0
_CHUNKS = 8


def _dcr_kernel(idxs_ref, seq_ref, mv_ref, ei_ref, a_s, b_s, n2_s, q_s):
    bi = pl.program_id(0)
    c = pl.program_id(1)
    C = pl.num_programs(1)
    CS = seq_ref.shape[1]
    H = seq_ref.shape[2]
    S = C * CS
    sep0 = idxs_ref[bi, 0]
    sep1 = idxs_ref[bi, 1]

    @pl.when(c == 0)
    def _():
        q_s[0:1, :] = seq_ref[0, 1:2, :]
        q_s[1:2, :] = seq_ref[0, pl.ds(sep0 - 1, 1), :]

    chunk = seq_ref[0]
    dn = (((1,), (1,)), ((), ()))
    ab = jax.lax.dot_general(q_s[...], chunk, dimension_numbers=dn,
                             preferred_element_type=jnp.float32)
    n2c = jax.lax.dot_general(jnp.ones((1, H), jnp.float32), chunk * chunk,
                              dimension_numbers=dn,
                              preferred_element_type=jnp.float32)
    a_s[pl.ds(c, 1), :] = ab[0:1, :]
    b_s[pl.ds(c, 1), :] = ab[1:2, :]
    n2_s[pl.ds(c, 1), :] = n2c

    @pl.when(c == C - 1)
    def _():
        q = q_s[...]
        qn = jnp.sqrt(jnp.sum(q * q))
        inv_qn = 1.0 / jnp.maximum(qn, _EPS)

        a2 = a_s[...]
        b2 = b_s[...]
        n2 = n2_s[...]
        pad_row = jnp.ones((1, CS), jnp.float32)
        b_dw = jnp.concatenate(
            [b2, jnp.concatenate([b2[1:, :], pad_row], axis=0)], axis=1)
        n2_dw = jnp.concatenate(
            [n2, jnp.concatenate([n2[1:, :], pad_row], axis=0)], axis=1)

        s_iota = jax.lax.broadcasted_iota(jnp.int32, (C, CS), 0)
        l_iota = jax.lax.broadcasted_iota(jnp.int32, (C, CS), 1)
        i_idx = s_iota * CS + l_iota

        mv = jnp.full((C, CS), _NEG, jnp.float32)
        best_o = jnp.zeros((C, CS), jnp.int32)
        for o in range(_MAX_ANS_LEN):
            b_o = jax.lax.slice(b_dw, (0, o), (C, o + CS))
            n2_o = jax.lax.slice(n2_dw, (0, o), (C, o + CS))
            num = a2 + b_o
            r = jnp.minimum(jax.lax.rsqrt(n2 + n2_o), 1.0 / _EPS)
            sim = num * r * inv_qn
            valid = i_idx < (sep1 - o)
            sim = jnp.where(valid, sim, _NEG)
            if o == 0:
                mv = sim
            else:
                upd = sim > mv
                mv = jnp.where(upd, sim, mv)
                best_o = jnp.where(upd, o, best_o)

        i_valid = (i_idx > sep0) & (i_idx < sep1)
        mv_ref[0] = jnp.where(i_valid, mv, _NEG)
        ei_ref[0] = jnp.where(i_valid, i_idx + best_o, -1)


@functools.partial(jax.jit, static_argnames=())
def kernel(sequence_outputs, idxs):
    B, S, H = sequence_outputs.shape
    C = _CHUNKS
    CS = S // C
    out_shape = (
        jax.ShapeDtypeStruct((B, C, CS), jnp.float32),
        jax.ShapeDtypeStruct((B, C, CS), jnp.int32),
    )
    mv, ei = pl.pallas_call(
        _dcr_kernel,
        grid=(B, C),
        in_specs=[
            pl.BlockSpec(memory_space=pltpu.SMEM),
            pl.BlockSpec((1, CS, H), lambda b, c: (b, c, 0)),
        ],
        out_specs=(
            pl.BlockSpec((1, C, CS), lambda b, c: (b, 0, 0)),
            pl.BlockSpec((1, C, CS), lambda b, c: (b, 0, 0)),
        ),
        out_shape=out_shape,
        scratch_shapes=[
            pltpu.VMEM((C, CS), jnp.float32),
            pltpu.VMEM((C, CS), jnp.float32),
            pltpu.VMEM((C, CS), jnp.float32),
            pltpu.VMEM((2, H), jnp.float32),
        ],
        compiler_params=pltpu.CompilerParams(
            dimension_semantics=("arbitrary", "arbitrary"),
        ),
    )(idxs, sequence_outputs)
    return mv.reshape(B, S), ei.reshape(B, S)


# speedup vs baseline: 1.8159x; 1.8159x over previous
import functools

import jax
import jax.numpy as jnp
from jax.experimental import pallas as pl
from jax.experimental.pallas import tpu as pltpu

_MAX_ANS_LEN = 30
_EPS = 1e-8
_NEG = -10000.0
_SPLIT = 4
_LANES = 256


def _dcr_kernel(idxs_ref, s0_ref, s1_ref, s2_ref, s3_ref, mv_ref, ei_ref):
    bi = pl.program_id(0)
    refs = (s0_ref, s1_ref, s2_ref, s3_ref)
    CS = s0_ref.shape[2]
    H = s0_ref.shape[3]
    C = _SPLIT * CS // _LANES
    sep0 = idxs_ref[bi, 0]
    sep1 = idxs_ref[bi, 1]

    q = jnp.concatenate(
        [s0_ref[0, 0, 1:2, :], s0_ref[0, 0, pl.ds(sep0 - 1, 1), :]], axis=0)

    dn = (((1,), (1,)), ((), ()))
    ones = jnp.ones((1, H), jnp.float32)
    rows_a, rows_b, rows_n = [], [], []
    for r in refs:
        chunk = r[0, 0]
        ab = jax.lax.dot_general(q, chunk, dimension_numbers=dn,
                                 preferred_element_type=jnp.float32)
        n2c = jax.lax.dot_general(ones, chunk * chunk, dimension_numbers=dn,
                                  preferred_element_type=jnp.float32)
        for j in range(CS // _LANES):
            rows_a.append(jax.lax.slice(ab, (0, j * _LANES), (1, (j + 1) * _LANES)))
            rows_b.append(jax.lax.slice(ab, (1, j * _LANES), (2, (j + 1) * _LANES)))
            rows_n.append(jax.lax.slice(n2c, (0, j * _LANES), (1, (j + 1) * _LANES)))
    a2 = jnp.concatenate(rows_a, axis=0)
    b2 = jnp.concatenate(rows_b, axis=0)
    n2 = jnp.concatenate(rows_n, axis=0)

    qn = jnp.sqrt(jnp.sum(q * q))
    inv_qn = 1.0 / jnp.maximum(qn, _EPS)

    pad_row = jnp.ones((1, _LANES), jnp.float32)
    b_dw = jnp.concatenate(
        [b2, jnp.concatenate([b2[1:, :], pad_row], axis=0)], axis=1)
    n2_dw = jnp.concatenate(
        [n2, jnp.concatenate([n2[1:, :], pad_row], axis=0)], axis=1)

    s_iota = jax.lax.broadcasted_iota(jnp.int32, (C, _LANES), 0)
    l_iota = jax.lax.broadcasted_iota(jnp.int32, (C, _LANES), 1)
    i_idx = s_iota * _LANES + l_iota

    mv = jnp.full((C, _LANES), _NEG, jnp.float32)
    best_o = jnp.zeros((C, _LANES), jnp.int32)
    for o in range(_MAX_ANS_LEN):
        b_o = jax.lax.slice(b_dw, (0, o), (C, o + _LANES))
        n2_o = jax.lax.slice(n2_dw, (0, o), (C, o + _LANES))
        num = a2 + b_o
        r = jnp.minimum(jax.lax.rsqrt(n2 + n2_o), 1.0 / _EPS)
        sim = num * r * inv_qn
        valid = i_idx < (sep1 - o)
        sim = jnp.where(valid, sim, _NEG)
        if o == 0:
            mv = sim
        else:
            upd = sim > mv
            mv = jnp.where(upd, sim, mv)
            best_o = jnp.where(upd, o, best_o)

    i_valid = (i_idx > sep0) & (i_idx < sep1)
    mv_ref[0] = jnp.where(i_valid, mv, _NEG)
    ei_ref[0] = jnp.where(i_valid, i_idx + best_o, -1)


@functools.partial(jax.jit, static_argnames=())
def kernel(sequence_outputs, idxs):
    B, S, H = sequence_outputs.shape
    CS = S // _SPLIT
    C = S // _LANES
    out_shape = (
        jax.ShapeDtypeStruct((B, C, _LANES), jnp.float32),
        jax.ShapeDtypeStruct((B, C, _LANES), jnp.int32),
    )
    seq4 = sequence_outputs.reshape(B, _SPLIT, CS, H)
    specs = [
        pl.BlockSpec((1, 1, CS, H), functools.partial(
            lambda k, b: (b, k, 0, 0), k))
        for k in range(_SPLIT)
    ]
    mv, ei = pl.pallas_call(
        _dcr_kernel,
        grid=(B,),
        in_specs=[pl.BlockSpec(memory_space=pltpu.SMEM)] + specs,
        out_specs=(
            pl.BlockSpec((1, C, _LANES), lambda b: (b, 0, 0)),
            pl.BlockSpec((1, C, _LANES), lambda b: (b, 0, 0)),
        ),
        out_shape=out_shape,
        compiler_params=pltpu.CompilerParams(
            dimension_semantics=("arbitrary",),
        ),
    )(idxs, seq4, seq4, seq4, seq4)
    return mv.reshape(B, S), ei.reshape(B, S)
